# baseline (speedup 1.0000x reference)
import functools

import jax
import jax.numpy as jnp
from jax import lax
from jax.experimental import pallas as pl
from jax.experimental.pallas import tpu as pltpu
from jax.experimental.pallas import tpu_sc as plsc

_N = 10000
_E = 320000
_NB = 10
_NBLK = _N // _NB
_EP = _E // 8
_EB = 10
_EBLK = _EP // _EB

_NW = 32
_EPW = _E // _NW
_S = 400
_NSC = _EPW // _S
_GC = 80
_NSUB = 16
_RPT = _N // _NSUB


def _node_embed_body(x_ref, wen_ref, ben_ref, a_ref, bm_ref,
                     xe_ref, ps_ref, pd_ref):
    xe = jnp.maximum(
        jnp.dot(x_ref[...], wen_ref[...], preferred_element_type=jnp.float32)
        + ben_ref[...], 0.0)
    xe_ref[...] = xe
    ps_ref[...] = jnp.dot(xe, a_ref[...], preferred_element_type=jnp.float32)
    pd_ref[...] = jnp.dot(xe, bm_ref[...], preferred_element_type=jnp.float32)


def _node_embed(x, W_en, b_en, A, Bm):
    return pl.pallas_call(
        _node_embed_body,
        grid=(_NB,),
        in_specs=[
            pl.BlockSpec((_NBLK, 128), lambda i: (i, 0)),
            pl.BlockSpec((128, 128), lambda i: (0, 0)),
            pl.BlockSpec((1, 128), lambda i: (0, 0)),
            pl.BlockSpec((128, 16), lambda i: (0, 0)),
            pl.BlockSpec((128, 16), lambda i: (0, 0)),
        ],
        out_specs=[
            pl.BlockSpec((_NBLK, 128), lambda i: (i, 0)),
            pl.BlockSpec((_NBLK, 16), lambda i: (i, 0)),
            pl.BlockSpec((_NBLK, 16), lambda i: (i, 0)),
        ],
        out_shape=[
            jax.ShapeDtypeStruct((_N, 128), jnp.float32),
            jax.ShapeDtypeStruct((_N, 16), jnp.float32),
            jax.ShapeDtypeStruct((_N, 16), jnp.float32),
        ],
    )(x, W_en, b_en, A, Bm)


def _edge_embed_body(ea_ref, wee_ref, bee_ref, c_ref, be_ref, eap_ref):
    ea = jnp.maximum(
        jnp.dot(ea_ref[...], wee_ref[...], preferred_element_type=jnp.float32)
        + bee_ref[...], 0.0)
    eap_ref[...] = jnp.dot(
        ea, c_ref[...], preferred_element_type=jnp.float32) + be_ref[...]


def _edge_embed_proj(edge_attr_p, K_wee, b_ee_t, K_c, b_e_t):
    return pl.pallas_call(
        _edge_embed_body,
        grid=(_EB,),
        in_specs=[
            pl.BlockSpec((_EBLK, 128), lambda i: (i, 0)),
            pl.BlockSpec((128, 128), lambda i: (0, 0)),
            pl.BlockSpec((1, 128), lambda i: (0, 0)),
            pl.BlockSpec((128, 128), lambda i: (0, 0)),
            pl.BlockSpec((1, 128), lambda i: (0, 0)),
        ],
        out_specs=pl.BlockSpec((_EBLK, 128), lambda i: (i, 0)),
        out_shape=jax.ShapeDtypeStruct((_EP, 128), jnp.float32),
        compiler_params=pltpu.CompilerParams(
            allow_input_fusion=[True, False, False, False, False]),
    )(edge_attr_p, K_wee, b_ee_t, K_c, b_e_t)


def _edge_proj_body(ea_ref, c_ref, be_ref, eap_ref):
    eap_ref[...] = jnp.dot(
        ea_ref[...], c_ref[...], preferred_element_type=jnp.float32) + be_ref[...]


def _edge_proj(ea_p, K_c, b_e_t):
    return pl.pallas_call(
        _edge_proj_body,
        grid=(_EB,),
        in_specs=[
            pl.BlockSpec((_EBLK, 128), lambda i: (i, 0)),
            pl.BlockSpec((128, 128), lambda i: (0, 0)),
            pl.BlockSpec((1, 128), lambda i: (0, 0)),
        ],
        out_specs=pl.BlockSpec((_EBLK, 128), lambda i: (i, 0)),
        out_shape=jax.ShapeDtypeStruct((_EP, 128), jnp.float32),
    )(ea_p, K_c, b_e_t)


def _node_update_body(xe_ref, agg_ref, wn1_ref, wn2_ref, bn_ref, a_ref, bm_ref,
                      xe2_ref, pool_ref, ps_ref, pd_ref):
    agg = agg_ref[0] + agg_ref[1]
    xe = jnp.maximum(
        jnp.dot(xe_ref[...], wn1_ref[...], preferred_element_type=jnp.float32)
        + jnp.dot(agg, wn2_ref[...], preferred_element_type=jnp.float32)
        + bn_ref[...], 0.0)
    xe2_ref[...] = xe
    pool_ref[...] = jnp.sum(xe, axis=-1, keepdims=True)
    ps_ref[...] = jnp.dot(xe, a_ref[...], preferred_element_type=jnp.float32)
    pd_ref[...] = jnp.dot(xe, bm_ref[...], preferred_element_type=jnp.float32)


def _node_update(xe, aggp, Wn1, Wn2, b_n, A, Bm):
    return pl.pallas_call(
        _node_update_body,
        grid=(_NB,),
        in_specs=[
            pl.BlockSpec((_NBLK, 128), lambda i: (i, 0)),
            pl.BlockSpec((2, _NBLK, 16), lambda i: (0, i, 0)),
            pl.BlockSpec((128, 128), lambda i: (0, 0)),
            pl.BlockSpec((16, 128), lambda i: (0, 0)),
            pl.BlockSpec((1, 128), lambda i: (0, 0)),
            pl.BlockSpec((128, 16), lambda i: (0, 0)),
            pl.BlockSpec((128, 16), lambda i: (0, 0)),
        ],
        out_specs=[
            pl.BlockSpec((_NBLK, 128), lambda i: (i, 0)),
            pl.BlockSpec((_NBLK, 1), lambda i: (i, 0)),
            pl.BlockSpec((_NBLK, 16), lambda i: (i, 0)),
            pl.BlockSpec((_NBLK, 16), lambda i: (i, 0)),
        ],
        out_shape=[
            jax.ShapeDtypeStruct((_N, 128), jnp.float32),
            jax.ShapeDtypeStruct((_N, 1), jnp.float32),
            jax.ShapeDtypeStruct((_N, 16), jnp.float32),
            jax.ShapeDtypeStruct((_N, 16), jnp.float32),
        ],
    )(xe, aggp, Wn1, Wn2, b_n, A, Bm)


def _node_final_body(xe_ref, agg_ref, wn1_ref, wn2_ref, bn_ref, pool_ref):
    agg = agg_ref[0] + agg_ref[1]
    xe = jnp.maximum(
        jnp.dot(xe_ref[...], wn1_ref[...], preferred_element_type=jnp.float32)
        + jnp.dot(agg, wn2_ref[...], preferred_element_type=jnp.float32)
        + bn_ref[...], 0.0)
    pool_ref[...] = jnp.sum(xe, axis=-1, keepdims=True)


def _node_final(xe, aggp, Wn1, Wn2, b_n):
    return pl.pallas_call(
        _node_final_body,
        grid=(_NB,),
        in_specs=[
            pl.BlockSpec((_NBLK, 128), lambda i: (i, 0)),
            pl.BlockSpec((2, _NBLK, 16), lambda i: (0, i, 0)),
            pl.BlockSpec((128, 128), lambda i: (0, 0)),
            pl.BlockSpec((16, 128), lambda i: (0, 0)),
            pl.BlockSpec((1, 128), lambda i: (0, 0)),
        ],
        out_specs=pl.BlockSpec((_NBLK, 1), lambda i: (i, 0)),
        out_shape=jax.ShapeDtypeStruct((_N, 1), jnp.float32),
    )(xe, aggp, Wn1, Wn2, b_n)


def _sc_body(write_ea, row_h, col_h, eap_h, ps_h, pd_h, *rest):
    if write_ea:
        ea_h, agg_h = rest[0], rest[1]
        scr = rest[2:]
    else:
        agg_h = rest[0]
        scr = rest[1:]
    (ridx0, cidx0, cidf0, eap0, src0, dst0, ea0,
     ridx1, cidx1, cidf1, eap1, src1, dst1, ea1,
     obuf, acc,
     sem_i0, sem_i1, sem_g0, sem_g1, sem_o0, sem_o1) = scr
    bufs = ((ridx0, cidx0, cidf0, eap0, src0, dst0, ea0, sem_i0, sem_g0, sem_o0),
            (ridx1, cidx1, cidf1, eap1, src1, dst1, ea1, sem_i1, sem_g1, sem_o1))
    cid = lax.axis_index("c")
    sid = lax.axis_index("s")
    w = cid * _NSUB + sid

    zero = jnp.zeros((16,), jnp.float32)

    def _zb(i, carry):
        obuf[i] = zero
        return carry

    lax.fori_loop(0, _RPT, _zb, 0)
    pltpu.sync_copy(obuf, acc.at[pl.ds(sid * _RPT, _RPT)])
    plsc.subcore_barrier()

    def fire_idx(c, b):
        ridx, cidx, cidf, eap_v, _, _, _, sem_i, _, _ = bufs[b]
        base = w * _EPW + c * _S
        gbase = base // _GC
        return [
            pltpu.async_copy(row_h.at[pl.ds(gbase, _S // _GC)], ridx, sem_i),
            pltpu.async_copy(col_h.at[pl.ds(gbase, _S // _GC)], cidx, sem_i),
            pltpu.async_copy(eap_h.at[pl.ds(base, _S)], eap_v, sem_i),
        ]

    def fire_gather(b):
        ridx, cidx, _, _, src_v, dst_v, _, _, sem_g, _ = bufs[b]
        ds = []
        for g in range(_S // _GC):
            sl = pl.ds(g * _GC, _GC)
            ds.append(pltpu.async_copy(ps_h.at[ridx.at[g]], src_v.at[sl], sem_g))
            ds.append(pltpu.async_copy(pd_h.at[cidx.at[g]], dst_v.at[sl], sem_g))
        return ds

    def compute(b):
        _, _, _, eap_v, src_v, dst_v, ea_v, _, _, _ = bufs[b]

        def _edges(e, cc):
            for u in range(8):
                i = e * 8 + u
                ea_v[i] = jnp.maximum(src_v[i] + dst_v[i] + eap_v[i], 0.0)
            return cc

        lax.fori_loop(0, _S // 8, _edges, 0)

    def fire_ea(c, b):
        _, _, _, _, _, _, ea_v, _, _, sem_o = bufs[b]
        if not write_ea:
            return []
        base = w * _EPW + c * _S
        return [pltpu.async_copy(ea_v, ea_h.at[pl.ds(base, _S)], sem_o)]

    def scatter(b):
        _, cidx, _, _, _, _, ea_v, _, _, _ = bufs[b]
        for g in range(_S // _GC):
            sl = pl.ds(g * _GC, _GC)
            pltpu.sync_copy(ea_v.at[sl], acc.at[cidx.at[g]], add=True)

    def _wait(ds):
        for d in ds:
            d.wait()

    def drain_idx(b):
        ridx, cidx, _, eap_v, _, _, _, sem_i, _, _ = bufs[b]
        pltpu.make_async_copy(row_h.at[pl.ds(0, _S // _GC)], ridx, sem_i).wait()
        pltpu.make_async_copy(col_h.at[pl.ds(0, _S // _GC)], cidx, sem_i).wait()
        pltpu.make_async_copy(eap_h.at[pl.ds(0, _S)], eap_v, sem_i).wait()

    def _pair(i, carry):
        a = 2 * i
        b = 2 * i + 1
        drain_idx(0)
        ga = fire_gather(0)
        drain_idx(1)
        gb = fire_gather(1)
        _wait(ga)
        compute(0)
        oa = fire_ea(a, 0)
        scatter(0)
        _wait(gb)
        compute(1)
        ob = fire_ea(b, 1)
        scatter(1)
        _wait(oa)
        fire_idx(a + 2, 0)
        _wait(ob)
        fire_idx(jnp.minimum(b + 2, _NSC - 1), 1)
        return carry

    fire_idx(0, 0)
    fire_idx(1, 1)
    lax.fori_loop(0, _NSC // 2, _pair, 0)
    drain_idx(0)
    _wait(fire_gather(0))
    compute(0)
    _wait(fire_ea(_NSC - 1, 0))
    scatter(0)
    drain_idx(1)
    plsc.subcore_barrier()

    @pl.when(sid == 0)
    def _():
        pltpu.sync_copy(acc, agg_h.at[cid])


def _make_sc_pass(write_ea):
    out_type = [jax.ShapeDtypeStruct((2, _N, 16), jnp.float32)]
    if write_ea:
        out_type = [jax.ShapeDtypeStruct((_E, 16), jnp.float32)] + out_type
    buf = [
        pltpu.VMEM((_S // _GC, _GC), jnp.int32),
        pltpu.VMEM((_S // _GC, _GC), jnp.int32),
        pltpu.VMEM((_S,), jnp.int32),
        pltpu.VMEM((_S, 16), jnp.float32),
        pltpu.VMEM((_S, 16), jnp.float32),
        pltpu.VMEM((_S, 16), jnp.float32),
        pltpu.VMEM((_S, 16), jnp.float32),
    ]
    return pl.kernel(
        functools.partial(_sc_body, write_ea),
        out_type=out_type,
        mesh=plsc.VectorSubcoreMesh(core_axis_name="c", subcore_axis_name="s"),
        scratch_types=buf + buf + [
            pltpu.VMEM((_RPT, 16), jnp.float32),
            pltpu.VMEM_SHARED((_N, 16), jnp.float32),
        ] + [pltpu.SemaphoreType.DMA] * 6,
        compiler_params=pltpu.CompilerParams(use_tc_tiling_on_sc=False),
        name="sc_edge_pass" + ("_wea" if write_ea else ""),
    )


_sc_pass_wea = _make_sc_pass(True)
_sc_pass_last = _make_sc_pass(False)


def kernel(x, edge_index, edge_attr, W_en, b_en, W_ee, b_ee, W_e, b_e, W_n, b_n):
    A = W_e[0:128]
    Bm = W_e[128:256]
    C = W_e[256:272]
    Wn1 = W_n[0:128]
    Wn2 = W_n[128:144]
    b_en2 = b_en.reshape(1, 128)
    b_n2 = b_n.reshape(1, 128)
    eye8 = jnp.eye(8, dtype=jnp.float32)
    K_wee = jnp.kron(eye8, W_ee)
    K_c = jnp.kron(eye8, C)
    b_ee_t = jnp.tile(b_ee, 8).reshape(1, 128)
    b_e_t = jnp.tile(b_e, 8).reshape(1, 128)
    row = edge_index[0].reshape(_E // _GC, _GC)
    col = edge_index[1].reshape(_E // _GC, _GC)

    xe, ps, pd = _node_embed(x, W_en, b_en2, A, Bm)
    eap = _edge_embed_proj(edge_attr.reshape(_EP, 128), K_wee, b_ee_t, K_c, b_e_t)

    ea, aggp = _sc_pass_wea(row, col, eap.reshape(_E, 16), ps, pd)
    xe, pool1, ps, pd = _node_update(xe, aggp, Wn1, Wn2, b_n2, A, Bm)

    eap = _edge_proj(ea.reshape(_EP, 128), K_c, b_e_t)
    (aggp,) = _sc_pass_last(row, col, eap.reshape(_E, 16), ps, pd)
    pool2 = _node_final(xe, aggp, Wn1, Wn2, b_n2)

    rep = jnp.concatenate([pool1, pool2], axis=-1)
    return (rep, rep)


# speedup vs baseline: 1.0138x; 1.0138x over previous
import functools

import jax
import jax.numpy as jnp
from jax import lax
from jax.experimental import pallas as pl
from jax.experimental.pallas import tpu as pltpu
from jax.experimental.pallas import tpu_sc as plsc

_N = 10000
_E = 320000
_NB = 10
_NBLK = _N // _NB
_EP = _E // 8
_EB = 10
_EBLK = _EP // _EB

_NW = 32
_EPW = _E // _NW
_S = 400
_NSC = _EPW // _S
_GC = 80
_NSUB = 16
_RPT = _N // _NSUB


def _node_embed_body(x_ref, wen_ref, ben_ref, a_ref, bm_ref,
                     xe_ref, ps_ref, pd_ref):
    xe = jnp.maximum(
        jnp.dot(x_ref[...], wen_ref[...], preferred_element_type=jnp.float32)
        + ben_ref[...], 0.0)
    xe_ref[...] = xe
    ps_ref[...] = jnp.dot(xe, a_ref[...], preferred_element_type=jnp.float32)
    pd_ref[...] = jnp.dot(xe, bm_ref[...], preferred_element_type=jnp.float32)


def _node_embed(x, W_en, b_en, A, Bm):
    return pl.pallas_call(
        _node_embed_body,
        grid=(_NB,),
        in_specs=[
            pl.BlockSpec((_NBLK, 128), lambda i: (i, 0)),
            pl.BlockSpec((128, 128), lambda i: (0, 0)),
            pl.BlockSpec((1, 128), lambda i: (0, 0)),
            pl.BlockSpec((128, 16), lambda i: (0, 0)),
            pl.BlockSpec((128, 16), lambda i: (0, 0)),
        ],
        out_specs=[
            pl.BlockSpec((_NBLK, 128), lambda i: (i, 0)),
            pl.BlockSpec((_NBLK, 16), lambda i: (i, 0)),
            pl.BlockSpec((_NBLK, 16), lambda i: (i, 0)),
        ],
        out_shape=[
            jax.ShapeDtypeStruct((_N, 128), jnp.float32),
            jax.ShapeDtypeStruct((_N, 16), jnp.float32),
            jax.ShapeDtypeStruct((_N, 16), jnp.float32),
        ],
    )(x, W_en, b_en, A, Bm)


def _edge_embed_body(ea_ref, wee_ref, bee_ref, c_ref, be_ref, eap_ref):
    ea = jnp.maximum(
        jnp.dot(ea_ref[...], wee_ref[...], preferred_element_type=jnp.float32)
        + bee_ref[...], 0.0)
    eap_ref[...] = jnp.dot(
        ea, c_ref[...], preferred_element_type=jnp.float32) + be_ref[...]


def _edge_embed_proj(edge_attr_p, K_wee, b_ee_t, K_c, b_e_t):
    return pl.pallas_call(
        _edge_embed_body,
        grid=(_EB,),
        in_specs=[
            pl.BlockSpec((_EBLK, 128), lambda i: (i, 0)),
            pl.BlockSpec((128, 128), lambda i: (0, 0)),
            pl.BlockSpec((1, 128), lambda i: (0, 0)),
            pl.BlockSpec((128, 128), lambda i: (0, 0)),
            pl.BlockSpec((1, 128), lambda i: (0, 0)),
        ],
        out_specs=pl.BlockSpec((_EBLK, 128), lambda i: (i, 0)),
        out_shape=jax.ShapeDtypeStruct((_EP, 128), jnp.float32),
        compiler_params=pltpu.CompilerParams(
            allow_input_fusion=[True, False, False, False, False]),
    )(edge_attr_p, K_wee, b_ee_t, K_c, b_e_t)


def _edge_proj_body(ea_ref, c_ref, be_ref, eap_ref):
    eap_ref[...] = jnp.dot(
        ea_ref[...], c_ref[...], preferred_element_type=jnp.float32) + be_ref[...]


def _edge_proj(ea_p, K_c, b_e_t):
    return pl.pallas_call(
        _edge_proj_body,
        grid=(_EB,),
        in_specs=[
            pl.BlockSpec((_EBLK, 128), lambda i: (i, 0)),
            pl.BlockSpec((128, 128), lambda i: (0, 0)),
            pl.BlockSpec((1, 128), lambda i: (0, 0)),
        ],
        out_specs=pl.BlockSpec((_EBLK, 128), lambda i: (i, 0)),
        out_shape=jax.ShapeDtypeStruct((_EP, 128), jnp.float32),
    )(ea_p, K_c, b_e_t)


def _node_update_body(xe_ref, agg_ref, wn1_ref, wn2_ref, bn_ref, a_ref, bm_ref,
                      ea_ref, c_ref, be_ref,
                      xe2_ref, pool_ref, ps_ref, pd_ref, eap_ref):
    agg = agg_ref[0] + agg_ref[1]
    xe = jnp.maximum(
        jnp.dot(xe_ref[...], wn1_ref[...], preferred_element_type=jnp.float32)
        + jnp.dot(agg, wn2_ref[...], preferred_element_type=jnp.float32)
        + bn_ref[...], 0.0)
    xe2_ref[...] = xe
    pool_ref[...] = jnp.sum(xe, axis=-1, keepdims=True)
    ps_ref[...] = jnp.dot(xe, a_ref[...], preferred_element_type=jnp.float32)
    pd_ref[...] = jnp.dot(xe, bm_ref[...], preferred_element_type=jnp.float32)
    eap_ref[...] = jnp.dot(
        ea_ref[...], c_ref[...], preferred_element_type=jnp.float32) + be_ref[...]


def _node_update(xe, aggp, Wn1, Wn2, b_n, A, Bm, ea_p, K_c, b_e_t):
    return pl.pallas_call(
        _node_update_body,
        grid=(_NB,),
        in_specs=[
            pl.BlockSpec((_NBLK, 128), lambda i: (i, 0)),
            pl.BlockSpec((2, _NBLK, 16), lambda i: (0, i, 0)),
            pl.BlockSpec((128, 128), lambda i: (0, 0)),
            pl.BlockSpec((16, 128), lambda i: (0, 0)),
            pl.BlockSpec((1, 128), lambda i: (0, 0)),
            pl.BlockSpec((128, 16), lambda i: (0, 0)),
            pl.BlockSpec((128, 16), lambda i: (0, 0)),
            pl.BlockSpec((_EBLK, 128), lambda i: (i, 0)),
            pl.BlockSpec((128, 128), lambda i: (0, 0)),
            pl.BlockSpec((1, 128), lambda i: (0, 0)),
        ],
        out_specs=[
            pl.BlockSpec((_NBLK, 128), lambda i: (i, 0)),
            pl.BlockSpec((_NBLK, 1), lambda i: (i, 0)),
            pl.BlockSpec((_NBLK, 16), lambda i: (i, 0)),
            pl.BlockSpec((_NBLK, 16), lambda i: (i, 0)),
            pl.BlockSpec((_EBLK, 128), lambda i: (i, 0)),
        ],
        out_shape=[
            jax.ShapeDtypeStruct((_N, 128), jnp.float32),
            jax.ShapeDtypeStruct((_N, 1), jnp.float32),
            jax.ShapeDtypeStruct((_N, 16), jnp.float32),
            jax.ShapeDtypeStruct((_N, 16), jnp.float32),
            jax.ShapeDtypeStruct((_EP, 128), jnp.float32),
        ],
    )(xe, aggp, Wn1, Wn2, b_n, A, Bm, ea_p, K_c, b_e_t)


def _node_final_body(xe_ref, agg_ref, wn1_ref, wn2_ref, bn_ref, pool_ref):
    agg = agg_ref[0] + agg_ref[1]
    xe = jnp.maximum(
        jnp.dot(xe_ref[...], wn1_ref[...], preferred_element_type=jnp.float32)
        + jnp.dot(agg, wn2_ref[...], preferred_element_type=jnp.float32)
        + bn_ref[...], 0.0)
    pool_ref[...] = jnp.sum(xe, axis=-1, keepdims=True)


def _node_final(xe, aggp, Wn1, Wn2, b_n):
    return pl.pallas_call(
        _node_final_body,
        grid=(_NB,),
        in_specs=[
            pl.BlockSpec((_NBLK, 128), lambda i: (i, 0)),
            pl.BlockSpec((2, _NBLK, 16), lambda i: (0, i, 0)),
            pl.BlockSpec((128, 128), lambda i: (0, 0)),
            pl.BlockSpec((16, 128), lambda i: (0, 0)),
            pl.BlockSpec((1, 128), lambda i: (0, 0)),
        ],
        out_specs=pl.BlockSpec((_NBLK, 1), lambda i: (i, 0)),
        out_shape=jax.ShapeDtypeStruct((_N, 1), jnp.float32),
    )(xe, aggp, Wn1, Wn2, b_n)


def _sc_body(write_ea, row_h, col_h, eap_h, ps_h, pd_h, *rest):
    if write_ea:
        ea_h, agg_h = rest[0], rest[1]
        scr = rest[2:]
    else:
        agg_h = rest[0]
        scr = rest[1:]
    (ridx0, cidx0, cidf0, eap0, src0, dst0, ea0,
     ridx1, cidx1, cidf1, eap1, src1, dst1, ea1,
     obuf, acc,
     sem_i0, sem_i1, sem_g0, sem_g1, sem_o0, sem_o1) = scr
    bufs = ((ridx0, cidx0, cidf0, eap0, src0, dst0, ea0, sem_i0, sem_g0, sem_o0),
            (ridx1, cidx1, cidf1, eap1, src1, dst1, ea1, sem_i1, sem_g1, sem_o1))
    cid = lax.axis_index("c")
    sid = lax.axis_index("s")
    w = cid * _NSUB + sid

    zero = jnp.zeros((16,), jnp.float32)

    def _zb(i, carry):
        obuf[i] = zero
        return carry

    lax.fori_loop(0, _RPT, _zb, 0)
    pltpu.sync_copy(obuf, acc.at[pl.ds(sid * _RPT, _RPT)])
    plsc.subcore_barrier()

    def fire_idx(c, b):
        ridx, cidx, cidf, eap_v, _, _, _, sem_i, _, _ = bufs[b]
        base = w * _EPW + c * _S
        gbase = base // _GC
        return [
            pltpu.async_copy(row_h.at[pl.ds(gbase, _S // _GC)], ridx, sem_i),
            pltpu.async_copy(col_h.at[pl.ds(gbase, _S // _GC)], cidx, sem_i),
            pltpu.async_copy(eap_h.at[pl.ds(base, _S)], eap_v, sem_i),
        ]

    def fire_gather(b):
        ridx, cidx, _, _, src_v, dst_v, _, _, sem_g, _ = bufs[b]
        ds = []
        for g in range(_S // _GC):
            sl = pl.ds(g * _GC, _GC)
            ds.append(pltpu.async_copy(ps_h.at[ridx.at[g]], src_v.at[sl], sem_g))
            ds.append(pltpu.async_copy(pd_h.at[cidx.at[g]], dst_v.at[sl], sem_g))
        return ds

    def compute(b):
        _, _, _, eap_v, src_v, dst_v, ea_v, _, _, _ = bufs[b]

        def _edges(e, cc):
            for u in range(8):
                i = e * 8 + u
                ea_v[i] = jnp.maximum(src_v[i] + dst_v[i] + eap_v[i], 0.0)
            return cc

        lax.fori_loop(0, _S // 8, _edges, 0)

    def fire_ea(c, b):
        _, _, _, _, _, _, ea_v, _, _, sem_o = bufs[b]
        if not write_ea:
            return []
        base = w * _EPW + c * _S
        return [pltpu.async_copy(ea_v, ea_h.at[pl.ds(base, _S)], sem_o)]

    def scatter(b):
        _, cidx, _, _, _, _, ea_v, _, _, _ = bufs[b]
        for g in range(_S // _GC):
            sl = pl.ds(g * _GC, _GC)
            pltpu.sync_copy(ea_v.at[sl], acc.at[cidx.at[g]], add=True)

    def _wait(ds):
        for d in ds:
            d.wait()

    def drain_idx(b):
        ridx, cidx, _, eap_v, _, _, _, sem_i, _, _ = bufs[b]
        pltpu.make_async_copy(row_h.at[pl.ds(0, _S // _GC)], ridx, sem_i).wait()
        pltpu.make_async_copy(col_h.at[pl.ds(0, _S // _GC)], cidx, sem_i).wait()
        pltpu.make_async_copy(eap_h.at[pl.ds(0, _S)], eap_v, sem_i).wait()

    def _pair(i, carry):
        a = 2 * i
        b = 2 * i + 1
        drain_idx(0)
        ga = fire_gather(0)
        drain_idx(1)
        gb = fire_gather(1)
        _wait(ga)
        compute(0)
        oa = fire_ea(a, 0)
        scatter(0)
        _wait(gb)
        compute(1)
        ob = fire_ea(b, 1)
        scatter(1)
        _wait(oa)
        fire_idx(a + 2, 0)
        _wait(ob)
        fire_idx(jnp.minimum(b + 2, _NSC - 1), 1)
        return carry

    fire_idx(0, 0)
    fire_idx(1, 1)
    lax.fori_loop(0, _NSC // 2, _pair, 0)
    drain_idx(0)
    _wait(fire_gather(0))
    compute(0)
    _wait(fire_ea(_NSC - 1, 0))
    scatter(0)
    drain_idx(1)
    plsc.subcore_barrier()

    @pl.when(sid == 0)
    def _():
        pltpu.sync_copy(acc, agg_h.at[cid])


def _make_sc_pass(write_ea):
    out_type = [jax.ShapeDtypeStruct((2, _N, 16), jnp.float32)]
    if write_ea:
        out_type = [jax.ShapeDtypeStruct((_E, 16), jnp.float32)] + out_type
    buf = [
        pltpu.VMEM((_S // _GC, _GC), jnp.int32),
        pltpu.VMEM((_S // _GC, _GC), jnp.int32),
        pltpu.VMEM((_S,), jnp.int32),
        pltpu.VMEM((_S, 16), jnp.float32),
        pltpu.VMEM((_S, 16), jnp.float32),
        pltpu.VMEM((_S, 16), jnp.float32),
        pltpu.VMEM((_S, 16), jnp.float32),
    ]
    return pl.kernel(
        functools.partial(_sc_body, write_ea),
        out_type=out_type,
        mesh=plsc.VectorSubcoreMesh(core_axis_name="c", subcore_axis_name="s"),
        scratch_types=buf + buf + [
            pltpu.VMEM((_RPT, 16), jnp.float32),
            pltpu.VMEM_SHARED((_N, 16), jnp.float32),
        ] + [pltpu.SemaphoreType.DMA] * 6,
        compiler_params=pltpu.CompilerParams(use_tc_tiling_on_sc=False),
        name="sc_edge_pass" + ("_wea" if write_ea else ""),
    )


_sc_pass_wea = _make_sc_pass(True)
_sc_pass_last = _make_sc_pass(False)


def kernel(x, edge_index, edge_attr, W_en, b_en, W_ee, b_ee, W_e, b_e, W_n, b_n):
    A = W_e[0:128]
    Bm = W_e[128:256]
    C = W_e[256:272]
    Wn1 = W_n[0:128]
    Wn2 = W_n[128:144]
    b_en2 = b_en.reshape(1, 128)
    b_n2 = b_n.reshape(1, 128)
    eye8 = jnp.eye(8, dtype=jnp.float32)
    K_wee = jnp.kron(eye8, W_ee)
    K_c = jnp.kron(eye8, C)
    b_ee_t = jnp.tile(b_ee, 8).reshape(1, 128)
    b_e_t = jnp.tile(b_e, 8).reshape(1, 128)
    row = edge_index[0].reshape(_E // _GC, _GC)
    col = edge_index[1].reshape(_E // _GC, _GC)

    xe, ps, pd = _node_embed(x, W_en, b_en2, A, Bm)
    eap = _edge_embed_proj(edge_attr.reshape(_EP, 128), K_wee, b_ee_t, K_c, b_e_t)

    ea, aggp = _sc_pass_wea(row, col, eap.reshape(_E, 16), ps, pd)
    xe, pool1, ps, pd, eap = _node_update(
        xe, aggp, Wn1, Wn2, b_n2, A, Bm, ea.reshape(_EP, 128), K_c, b_e_t)

    (aggp,) = _sc_pass_last(row, col, eap.reshape(_E, 16), ps, pd)
    pool2 = _node_final(xe, aggp, Wn1, Wn2, b_n2)

    rep = jnp.concatenate([pool1, pool2], axis=-1)
    return (rep, rep)
